# baseline (speedup 1.0000x reference)
import jax
import jax.numpy as jnp
from jax.experimental import pallas as pl

N_TOK = 8192
D = 4096
E = 64
TOP_K = 8
BLK = 256
WCOLS = 256
INT_MIN = -2147483648


def _router_kernel(x_ref, wcat_ref, bcat_ref, eps_ref,
                   router_ref, idx_ref, skip_ref):
    x = x_ref[...]
    out = jax.lax.dot_general(
        x, wcat_ref[...], (((1,), (0,)), ((), ())),
        preferred_element_type=jnp.float32) + bcat_ref[...]
    logits = out[:, :E]
    noise_logits = out[:, E:2 * E]
    skip_logits = out[:, 2 * E:2 * E + 1]

    nl = logits + eps_ref[...] * jax.nn.softplus(noise_logits)

    bits = jax.lax.bitcast_convert_type(nl, jnp.int32)
    key = jnp.where(bits >= 0, bits, bits ^ jnp.int32(0x7FFFFFFF))
    iota = jax.lax.broadcasted_iota(jnp.int32, (BLK, E), 1)
    key = (key & jnp.int32(~63)) | (jnp.int32(E - 1) - iota)

    idxs = []
    cur = key
    top_key = None
    for _ in range(TOP_K):
        m = jnp.max(cur, axis=1, keepdims=True)
        if top_key is None:
            top_key = m
        idx = jnp.int32(E - 1) - (m & jnp.int32(63))
        idxs.append(idx)
        cur = jnp.where(iota == idx, jnp.int32(INT_MIN), cur)
    idx_ref[...] = jnp.concatenate(idxs, axis=1)

    mbits = top_key & jnp.int32(~63)
    mbits = jnp.where(mbits >= 0, mbits, mbits ^ jnp.int32(0x7FFFFFFF))
    vmax = jax.lax.bitcast_convert_type(mbits, jnp.float32)

    selected = cur == jnp.int32(INT_MIN)
    p = jnp.where(selected, jnp.exp(nl - vmax), 0.0)
    denom = jnp.sum(p, axis=1, keepdims=True)
    router_ref[...] = p / denom

    skip_ref[...] = jax.nn.sigmoid(skip_logits)


def kernel(x, W_router, b_router, W_noise, b_noise, W_skip, b_skip):
    eps = jax.random.normal(jax.random.key(42), (N_TOK, E), jnp.float32)

    wcat = jnp.concatenate(
        [W_router, W_noise, W_skip,
         jnp.zeros((D, WCOLS - 2 * E - 1), jnp.float32)], axis=1)
    bcat = jnp.concatenate(
        [b_router, b_noise, b_skip,
         jnp.zeros((WCOLS - 2 * E - 1,), jnp.float32)])[None, :]

    grid = N_TOK // BLK
    router_out, indices, skip_prob = pl.pallas_call(
        _router_kernel,
        grid=(grid,),
        in_specs=[
            pl.BlockSpec((BLK, D), lambda i: (i, 0)),
            pl.BlockSpec((D, WCOLS), lambda i: (0, 0)),
            pl.BlockSpec((1, WCOLS), lambda i: (0, 0)),
            pl.BlockSpec((BLK, E), lambda i: (i, 0)),
        ],
        out_specs=[
            pl.BlockSpec((BLK, E), lambda i: (i, 0)),
            pl.BlockSpec((BLK, TOP_K), lambda i: (i, 0)),
            pl.BlockSpec((BLK, 1), lambda i: (i, 0)),
        ],
        out_shape=[
            jax.ShapeDtypeStruct((N_TOK, E), jnp.float32),
            jax.ShapeDtypeStruct((N_TOK, TOP_K), jnp.int32),
            jax.ShapeDtypeStruct((N_TOK, 1), jnp.float32),
        ],
    )(x, wcat, bcat, eps)
    return router_out, indices, skip_prob


# speedup vs baseline: 1.5157x; 1.5157x over previous
import jax
import jax.numpy as jnp
from jax.experimental import pallas as pl

N_TOK = 8192
D = 4096
E = 64
TOP_K = 8
BLK = 512
WCOLS = 256
INT_MIN = -2147483648


def _router_kernel(x_ref, wcat_ref, bcat_ref, eps_ref,
                   router_ref, idx_ref, skip_ref):
    x = x_ref[...]
    out = jax.lax.dot_general(
        x, wcat_ref[...], (((1,), (0,)), ((), ())),
        preferred_element_type=jnp.float32) + bcat_ref[...]
    logits = out[:, :E]
    noise_logits = out[:, E:2 * E]
    skip_logits = out[:, 2 * E:2 * E + 1]

    nl = logits + eps_ref[...] * jax.nn.softplus(noise_logits)

    bits = jax.lax.bitcast_convert_type(nl, jnp.int32)
    key = jnp.where(bits >= 0, bits, bits ^ jnp.int32(0x7FFFFFFF))
    iota = jax.lax.broadcasted_iota(jnp.int32, (BLK, E), 1)
    key = (key & jnp.int32(~63)) | (jnp.int32(E - 1) - iota)

    idxs = []
    cur = key
    top_key = None
    for _ in range(TOP_K):
        m = jnp.max(cur, axis=1, keepdims=True)
        if top_key is None:
            top_key = m
        idx = jnp.int32(E - 1) - (m & jnp.int32(63))
        idxs.append(idx)
        cur = jnp.where(iota == idx, jnp.int32(INT_MIN), cur)
    idx_ref[...] = jnp.concatenate(idxs, axis=1)

    mbits = top_key & jnp.int32(~63)
    mbits = jnp.where(mbits >= 0, mbits, mbits ^ jnp.int32(0x7FFFFFFF))
    vmax = jax.lax.bitcast_convert_type(mbits, jnp.float32)

    selected = cur == jnp.int32(INT_MIN)
    p = jnp.where(selected, jnp.exp(nl - vmax), 0.0)
    denom = jnp.sum(p, axis=1, keepdims=True)
    router_ref[...] = p / denom

    skip_ref[...] = jax.nn.sigmoid(skip_logits)


def kernel(x, W_router, b_router, W_noise, b_noise, W_skip, b_skip):
    with jax.ensure_compile_time_eval():
        eps = jax.random.normal(jax.random.key(42), (N_TOK, E), jnp.float32)

    wcat = jnp.concatenate(
        [W_router, W_noise, W_skip,
         jnp.zeros((D, WCOLS - 2 * E - 1), jnp.float32)], axis=1)
    bcat = jnp.concatenate(
        [b_router, b_noise, b_skip,
         jnp.zeros((WCOLS - 2 * E - 1,), jnp.float32)])[None, :]

    grid = N_TOK // BLK
    router_out, indices, skip_prob = pl.pallas_call(
        _router_kernel,
        grid=(grid,),
        in_specs=[
            pl.BlockSpec((BLK, D), lambda i: (i, 0)),
            pl.BlockSpec((D, WCOLS), lambda i: (0, 0)),
            pl.BlockSpec((1, WCOLS), lambda i: (0, 0)),
            pl.BlockSpec((BLK, E), lambda i: (i, 0)),
        ],
        out_specs=[
            pl.BlockSpec((BLK, E), lambda i: (i, 0)),
            pl.BlockSpec((BLK, TOP_K), lambda i: (i, 0)),
            pl.BlockSpec((BLK, 1), lambda i: (i, 0)),
        ],
        out_shape=[
            jax.ShapeDtypeStruct((N_TOK, E), jnp.float32),
            jax.ShapeDtypeStruct((N_TOK, TOP_K), jnp.int32),
            jax.ShapeDtypeStruct((N_TOK, 1), jnp.float32),
        ],
    )(x, wcat, bcat, eps)
    return router_out, indices, skip_prob


# speedup vs baseline: 1.5842x; 1.0452x over previous
import jax
import jax.numpy as jnp
from jax.experimental import pallas as pl

N_TOK = 8192
D = 4096
E = 64
TOP_K = 8
BLK = 1024
WCOLS = 256
INT_MIN = -2147483648


def _router_kernel(x_ref, wcat_ref, bcat_ref, eps_ref,
                   router_ref, idx_ref, skip_ref):
    x = x_ref[...]
    out = jax.lax.dot_general(
        x, wcat_ref[...], (((1,), (0,)), ((), ())),
        preferred_element_type=jnp.float32) + bcat_ref[...]
    logits = out[:, :E]
    noise_logits = out[:, E:2 * E]
    skip_logits = out[:, 2 * E:2 * E + 1]

    nl = logits + eps_ref[...] * jax.nn.softplus(noise_logits)

    bits = jax.lax.bitcast_convert_type(nl, jnp.int32)
    key = jnp.where(bits >= 0, bits, bits ^ jnp.int32(0x7FFFFFFF))
    iota = jax.lax.broadcasted_iota(jnp.int32, (BLK, E), 1)
    key = (key & jnp.int32(~63)) | (jnp.int32(E - 1) - iota)

    idxs = []
    cur = key
    top_key = None
    for _ in range(TOP_K):
        m = jnp.max(cur, axis=1, keepdims=True)
        if top_key is None:
            top_key = m
        idx = jnp.int32(E - 1) - (m & jnp.int32(63))
        idxs.append(idx)
        cur = jnp.where(iota == idx, jnp.int32(INT_MIN), cur)
    idx_ref[...] = jnp.concatenate(idxs, axis=1)

    mbits = top_key & jnp.int32(~63)
    mbits = jnp.where(mbits >= 0, mbits, mbits ^ jnp.int32(0x7FFFFFFF))
    vmax = jax.lax.bitcast_convert_type(mbits, jnp.float32)

    selected = cur == jnp.int32(INT_MIN)
    p = jnp.where(selected, jnp.exp(nl - vmax), 0.0)
    denom = jnp.sum(p, axis=1, keepdims=True)
    router_ref[...] = p / denom

    skip_ref[...] = jax.nn.sigmoid(skip_logits)


def kernel(x, W_router, b_router, W_noise, b_noise, W_skip, b_skip):
    with jax.ensure_compile_time_eval():
        eps = jax.random.normal(jax.random.key(42), (N_TOK, E), jnp.float32)

    wcat = jnp.concatenate(
        [W_router, W_noise, W_skip,
         jnp.zeros((D, WCOLS - 2 * E - 1), jnp.float32)], axis=1)
    bcat = jnp.concatenate(
        [b_router, b_noise, b_skip,
         jnp.zeros((WCOLS - 2 * E - 1,), jnp.float32)])[None, :]

    grid = N_TOK // BLK
    router_out, indices, skip_prob = pl.pallas_call(
        _router_kernel,
        grid=(grid,),
        in_specs=[
            pl.BlockSpec((BLK, D), lambda i: (i, 0)),
            pl.BlockSpec((D, WCOLS), lambda i: (0, 0)),
            pl.BlockSpec((1, WCOLS), lambda i: (0, 0)),
            pl.BlockSpec((BLK, E), lambda i: (i, 0)),
        ],
        out_specs=[
            pl.BlockSpec((BLK, E), lambda i: (i, 0)),
            pl.BlockSpec((BLK, TOP_K), lambda i: (i, 0)),
            pl.BlockSpec((BLK, 1), lambda i: (i, 0)),
        ],
        out_shape=[
            jax.ShapeDtypeStruct((N_TOK, E), jnp.float32),
            jax.ShapeDtypeStruct((N_TOK, TOP_K), jnp.int32),
            jax.ShapeDtypeStruct((N_TOK, 1), jnp.float32),
        ],
    )(x, wcat, bcat, eps)
    return router_out, indices, skip_prob


# speedup vs baseline: 2.1267x; 1.3425x over previous
import jax
import jax.numpy as jnp
from jax.experimental import pallas as pl

N_TOK = 8192
D = 4096
E = 64
TOP_K = 8
BLK = 1024
WCOLS = 256
INT_MIN = -2147483648


def _router_kernel(xt_ref, wcat_ref, bcat_ref, eps_ref,
                   router_ref, idx_ref, skip_ref):
    out = jax.lax.dot_general(
        wcat_ref[...], xt_ref[...], (((0,), (1,)), ((), ())),
        preferred_element_type=jnp.float32) + bcat_ref[...]
    logits = out[:E, :]
    noise_logits = out[E:2 * E, :]
    skip_logits = out[2 * E:2 * E + 1, :]

    nl = logits + eps_ref[...] * jax.nn.softplus(noise_logits)

    bits = jax.lax.bitcast_convert_type(nl, jnp.int32)
    key = jnp.where(bits >= 0, bits, bits ^ jnp.int32(0x7FFFFFFF))
    iota = jax.lax.broadcasted_iota(jnp.int32, (E, BLK), 0)
    key = (key & jnp.int32(~63)) | (jnp.int32(E - 1) - iota)

    idxs = []
    cur = key
    top_key = None
    for _ in range(TOP_K):
        m = jnp.max(cur, axis=0, keepdims=True)
        if top_key is None:
            top_key = m
        idx = jnp.int32(E - 1) - (m & jnp.int32(63))
        idxs.append(idx)
        cur = jnp.where(iota == idx, jnp.int32(INT_MIN), cur)
    idx_ref[...] = jnp.concatenate(idxs, axis=0)

    mbits = top_key & jnp.int32(~63)
    mbits = jnp.where(mbits >= 0, mbits, mbits ^ jnp.int32(0x7FFFFFFF))
    vmax = jax.lax.bitcast_convert_type(mbits, jnp.float32)

    selected = cur == jnp.int32(INT_MIN)
    p = jnp.where(selected, jnp.exp(nl - vmax), 0.0)
    denom = jnp.sum(p, axis=0, keepdims=True)
    router_ref[...] = p / denom

    skip_ref[...] = jax.nn.sigmoid(skip_logits)


def kernel(x, W_router, b_router, W_noise, b_noise, W_skip, b_skip):
    with jax.ensure_compile_time_eval():
        eps_t = jax.random.normal(
            jax.random.key(42), (N_TOK, E), jnp.float32).T

    wcat = jnp.concatenate(
        [W_router, W_noise, W_skip,
         jnp.zeros((D, WCOLS - 2 * E - 1), jnp.float32)], axis=1)
    bcat = jnp.concatenate(
        [b_router, b_noise, b_skip,
         jnp.zeros((WCOLS - 2 * E - 1,), jnp.float32)])[:, None]

    grid = N_TOK // BLK
    router_t, idx_t, skip_t = pl.pallas_call(
        _router_kernel,
        grid=(grid,),
        in_specs=[
            pl.BlockSpec((BLK, D), lambda i: (i, 0)),
            pl.BlockSpec((D, WCOLS), lambda i: (0, 0)),
            pl.BlockSpec((WCOLS, 1), lambda i: (0, 0)),
            pl.BlockSpec((E, BLK), lambda i: (0, i)),
        ],
        out_specs=[
            pl.BlockSpec((E, BLK), lambda i: (0, i)),
            pl.BlockSpec((TOP_K, BLK), lambda i: (0, i)),
            pl.BlockSpec((1, BLK), lambda i: (0, i)),
        ],
        out_shape=[
            jax.ShapeDtypeStruct((E, N_TOK), jnp.float32),
            jax.ShapeDtypeStruct((TOP_K, N_TOK), jnp.int32),
            jax.ShapeDtypeStruct((1, N_TOK), jnp.float32),
        ],
    )(x, wcat, bcat, eps_t)
    return router_t.T, idx_t.T, skip_t.T
